# Initial kernel scaffold; baseline (speedup 1.0000x reference)
import jax
import jax.numpy as jnp
from jax.experimental import pallas as pl


def kernel(node, edge, edge_index, batch, g, e_W1, e_b1, e_W2, e_b2, n_W1, n_b1, n_W2, n_b2, g_W1, g_b1, g_W2, g_b2):
    raise NotImplementedError("write your pallas kernel here")


# speedup vs baseline: 6.6835x; 6.6835x over previous
import functools

import jax
import jax.numpy as jnp
from jax import lax
from jax.experimental import pallas as pl
from jax.experimental.pallas import tpu as pltpu
from jax.experimental.pallas import tpu_sc as plsc

N_NODES = 10000
N_EDGES = 320000
N_GRAPHS = 16
NODE_C = 128
EDGE_C = 16
MID_C = 128
GLOB_C = 16

NC = 2
NS = 16
NW = NC * NS
EPT = N_EDGES // NW
KCH = 128
RPT_STRIDE = 624
RPT = 640


def _p_kernel(node_ref, b2d_ref, g_ref, w1a_ref, w1b_ref, w1d_ref, b1_ref,
              pa_ref, pb_ref):
    n = node_ref[:]
    g2e = jnp.dot(g_ref[:], w1d_ref[:], preferred_element_type=jnp.float32)
    g2e = g2e + b1_ref[:]
    onehot = (b2d_ref[:] == lax.broadcasted_iota(jnp.int32, (1, N_GRAPHS), 1)
              ).astype(jnp.float32)
    pa = jnp.dot(n, w1a_ref[:], preferred_element_type=jnp.float32)
    pa_ref[:] = pa + jnp.dot(onehot, g2e, preferred_element_type=jnp.float32)
    pb_ref[:] = jnp.dot(n, w1b_ref[:], preferred_element_type=jnp.float32)


def _stage_p(node, batch2d, g, w1a, w1b, w1d, b1):
    bp = 2000
    grid = N_NODES // bp
    return pl.pallas_call(
        _p_kernel,
        grid=(grid,),
        in_specs=[
            pl.BlockSpec((bp, NODE_C), lambda i: (i, 0)),
            pl.BlockSpec((bp, 1), lambda i: (i, 0)),
            pl.BlockSpec((N_GRAPHS, GLOB_C), lambda i: (0, 0)),
            pl.BlockSpec((NODE_C, MID_C), lambda i: (0, 0)),
            pl.BlockSpec((NODE_C, MID_C), lambda i: (0, 0)),
            pl.BlockSpec((GLOB_C, MID_C), lambda i: (0, 0)),
            pl.BlockSpec((1, MID_C), lambda i: (0, 0)),
        ],
        out_specs=[
            pl.BlockSpec((bp, MID_C), lambda i: (i, 0)),
            pl.BlockSpec((bp, MID_C), lambda i: (i, 0)),
        ],
        out_shape=[
            jax.ShapeDtypeStruct((N_NODES, MID_C), jnp.float32),
            jax.ShapeDtypeStruct((N_NODES, MID_C), jnp.float32),
        ],
    )(node, batch2d, g, w1a, w1b, w1d, b1)


def _stage_g(pa, pb, rol, col):
    mesh = plsc.VectorSubcoreMesh(core_axis_name="c", subcore_axis_name="s")
    n_chunks = (EPT + KCH - 1) // KCH
    n_pairs = n_chunks // 2

    @functools.partial(
        pl.kernel,
        mesh=mesh,
        out_type=jax.ShapeDtypeStruct((N_EDGES, MID_C), jnp.float32),
        scratch_types=[
            pltpu.VMEM((KCH,), jnp.int32),
            pltpu.VMEM((KCH,), jnp.int32),
            pltpu.VMEM((KCH,), jnp.int32),
            pltpu.VMEM((KCH,), jnp.int32),
            pltpu.VMEM((KCH, MID_C), jnp.float32),
            pltpu.VMEM((KCH, MID_C), jnp.float32),
            pltpu.VMEM((KCH, MID_C), jnp.float32),
            pltpu.VMEM((KCH, MID_C), jnp.float32),
            pltpu.VMEM((KCH, MID_C), jnp.float32),
            pltpu.SemaphoreType.DMA,
            pltpu.SemaphoreType.DMA,
            pltpu.SemaphoreType.DMA,
            pltpu.SemaphoreType.DMA,
            pltpu.SemaphoreType.DMA,
            pltpu.SemaphoreType.DMA,
        ],
    )
    def k(pa_hbm, pb_hbm, rol_hbm, col_hbm, out_hbm,
          rol0, col0, rol1, col1, bufa0, bufb0, bufa1, bufb1, bufo,
          ga0, gb0, ga1, gb1, ia, ib):
        wid = lax.axis_index("s") * NC + lax.axis_index("c")
        base = wid * EPT

        def coff(c):
            return base + jnp.minimum(c * KCH, EPT - KCH)

        def load_idx(c, rv, cv):
            cpa = pltpu.async_copy(rol_hbm.at[pl.ds(coff(c), KCH)], rv, ia)
            cpb = pltpu.async_copy(col_hbm.at[pl.ds(coff(c), KCH)], cv, ib)
            cpa.wait()
            cpb.wait()

        def fire(rv, cv, ba, bb, sa, sb):
            return (pltpu.async_copy(pa_hbm.at[rv], ba, sa),
                    pltpu.async_copy(pb_hbm.at[cv], bb, sb))

        def drain(c, cps, ba, bb):
            cps[0].wait()
            cps[1].wait()

            def row(i, c2):
                for j in range(MID_C // 16):
                    sl = pl.ds(j * 16, 16)
                    bufo[i, sl] = ba[i, sl] + bb[i, sl]
                return c2

            lax.fori_loop(0, KCH, row, 0)
            pltpu.sync_copy(bufo, out_hbm.at[pl.ds(coff(c), KCH)])

        load_idx(0, rol0, col0)
        fire(rol0, col0, bufa0, bufb0, ga0, gb0)

        def pair(cc, carry):
            c0 = 2 * cc
            load_idx(c0 + 1, rol1, col1)
            cps1 = fire(rol1, col1, bufa1, bufb1, ga1, gb1)
            drain(c0, (pltpu.make_async_copy(pa_hbm.at[rol0], bufa0, ga0),
                       pltpu.make_async_copy(pb_hbm.at[col0], bufb0, gb0)),
                  bufa0, bufb0)
            load_idx(c0 + 2, rol0, col0)
            fire(rol0, col0, bufa0, bufb0, ga0, gb0)
            drain(c0 + 1, cps1, bufa1, bufb1)
            return carry

        lax.fori_loop(0, n_pairs, pair, 0)
        drain(n_chunks - 1,
              (pltpu.make_async_copy(pa_hbm.at[rol0], bufa0, ga0),
               pltpu.make_async_copy(pb_hbm.at[col0], bufb0, gb0)),
              bufa0, bufb0)

    return k(pa, pb, rol, col)


def _e_kernel(acc_ref, edge_ref, w1c_ref, w2_ref, b2_ref, out_ref):
    h = acc_ref[:] + jnp.dot(edge_ref[:], w1c_ref[:],
                             preferred_element_type=jnp.float32)
    h = jnp.maximum(h, 0.0)
    out_ref[:] = jnp.dot(h, w2_ref[:],
                         preferred_element_type=jnp.float32) + b2_ref[:]


def _stage_e(acc, edge, w1c, w2, b2):
    be = 2000
    grid = N_EDGES // be
    return pl.pallas_call(
        _e_kernel,
        grid=(grid,),
        in_specs=[
            pl.BlockSpec((be, MID_C), lambda i: (i, 0)),
            pl.BlockSpec((be, EDGE_C), lambda i: (i, 0)),
            pl.BlockSpec((EDGE_C, MID_C), lambda i: (0, 0)),
            pl.BlockSpec((MID_C, EDGE_C), lambda i: (0, 0)),
            pl.BlockSpec((1, EDGE_C), lambda i: (0, 0)),
        ],
        out_specs=pl.BlockSpec((be, EDGE_C), lambda i: (i, 0)),
        out_shape=jax.ShapeDtypeStruct((N_EDGES, EDGE_C), jnp.float32),
    )(acc, edge, w1c, w2, b2)


NHALF = N_NODES // NC
EPT2 = N_EDGES // NS


def _stage_s(rol, edge_new_flat):
    mesh = plsc.VectorSubcoreMesh(core_axis_name="c", subcore_axis_name="s")
    n_full = EPT2 // KCH
    tail = EPT2 - n_full * KCH

    @functools.partial(
        pl.kernel,
        mesh=mesh,
        out_type=jax.ShapeDtypeStruct((NS * N_NODES * EDGE_C,), jnp.float32),
        compiler_params=pltpu.CompilerParams(needs_layout_passes=False),
        scratch_types=[
            pltpu.VMEM((KCH,), jnp.int32),
            pltpu.VMEM((KCH * EDGE_C,), jnp.float32),
            pltpu.VMEM((NHALF * EDGE_C,), jnp.float32),
            pltpu.SemaphoreType.DMA,
            pltpu.SemaphoreType.DMA,
        ],
    )
    def k(rol_hbm, en_hbm, out_hbm, rol_v, val_v, vtab, sema, semb):
        cid = lax.axis_index("c")
        sid = lax.axis_index("s")
        ebase = sid * EPT2
        nbase = cid * NHALF

        def zrow(i, carry):
            vtab[pl.ds(i * 16, 16)] = jnp.zeros((16,), jnp.float32)
            return carry

        lax.fori_loop(0, NHALF, zrow, 0)

        def do_chunk(off, klen):
            cpa = pltpu.async_copy(rol_hbm.at[pl.ds(off, klen)],
                                   rol_v.at[pl.ds(0, klen)], sema)
            cpb = pltpu.async_copy(
                en_hbm.at[pl.ds(off * EDGE_C, klen * EDGE_C)],
                val_v.at[pl.ds(0, klen * EDGE_C)], semb)
            cpa.wait()
            cpb.wait()
            for g in range(klen // 16):
                rv = rol_v[pl.ds(g * 16, 16)]
                for l in range(16):
                    ei = g * 16 + l
                    loc = rv[l] - nbase

                    @pl.when((loc >= 0) & (loc < NHALF))
                    def _(ei=ei, loc=loc):
                        plsc.addupdate(vtab.at[pl.ds(loc * EDGE_C, EDGE_C)],
                                       val_v[pl.ds(ei * EDGE_C, EDGE_C)])

        def chunk(c, carry):
            do_chunk(ebase + c * KCH, KCH)
            return carry

        lax.fori_loop(0, n_full, chunk, 0)
        do_chunk(ebase + n_full * KCH, tail)

        pltpu.sync_copy(vtab,
                        out_hbm.at[pl.ds((sid * N_NODES + cid * NHALF)
                                         * EDGE_C, NHALF * EDGE_C)])

    return k(rol, edge_new_flat)


def _n_kernel(node_ref, aggp_ref, b2d_ref, g_ref,
              nw1a_ref, nw1b_ref, nw1c_ref, nb1_ref, nw2_ref, nb2_ref,
              gw1a_ref, gw1b_ref, gw1c_ref, gb1_ref, gw2_ref, gb2_ref,
              nn_ref, gnew_ref, nagg_s, eagg_s):
    i = pl.program_id(0)
    nblocks = pl.num_programs(0)
    agg = jnp.sum(aggp_ref[:], axis=0)
    onehot = (b2d_ref[:] == lax.broadcasted_iota(jnp.int32, (1, N_GRAPHS), 1)
              ).astype(jnp.float32)
    g2n = jnp.dot(g_ref[:], nw1c_ref[:],
                  preferred_element_type=jnp.float32) + nb1_ref[:]
    h = (jnp.dot(node_ref[:], nw1a_ref[:], preferred_element_type=jnp.float32)
         + jnp.dot(agg, nw1b_ref[:], preferred_element_type=jnp.float32)
         + jnp.dot(onehot, g2n, preferred_element_type=jnp.float32))
    h = jnp.maximum(h, 0.0)
    nn = jnp.dot(h, nw2_ref[:], preferred_element_type=jnp.float32) + nb2_ref[:]
    nn_ref[:] = nn

    dn = (((0,), (0,)), ((), ()))
    nagg_c = lax.dot_general(onehot, nn, dn, preferred_element_type=jnp.float32)
    eagg_c = lax.dot_general(onehot, agg, dn, preferred_element_type=jnp.float32)

    @pl.when(i == 0)
    def _():
        nagg_s[:] = nagg_c
        eagg_s[:] = eagg_c

    @pl.when(i > 0)
    def _():
        nagg_s[:] = nagg_s[:] + nagg_c
        eagg_s[:] = eagg_s[:] + eagg_c

    @pl.when(i == nblocks - 1)
    def _():
        hg = (jnp.dot(nagg_s[:], gw1a_ref[:], preferred_element_type=jnp.float32)
              + jnp.dot(eagg_s[:], gw1b_ref[:], preferred_element_type=jnp.float32)
              + jnp.dot(g_ref[:], gw1c_ref[:], preferred_element_type=jnp.float32)
              + gb1_ref[:])
        hg = jnp.maximum(hg, 0.0)
        gnew_ref[:] = jnp.dot(hg, gw2_ref[:],
                              preferred_element_type=jnp.float32) + gb2_ref[:]


def _stage_n(node, aggp, batch2d, g,
             nw1a, nw1b, nw1c, nb1, nw2, nb2,
             gw1a, gw1b, gw1c, gb1, gw2, gb2):
    bn = 2000
    grid = N_NODES // bn
    full = lambda r, c: pl.BlockSpec((r, c), lambda i: (0, 0))
    return pl.pallas_call(
        _n_kernel,
        grid=(grid,),
        in_specs=[
            pl.BlockSpec((bn, NODE_C), lambda i: (i, 0)),
            pl.BlockSpec((NS, bn, EDGE_C), lambda i: (0, i, 0)),
            pl.BlockSpec((bn, 1), lambda i: (i, 0)),
            full(N_GRAPHS, GLOB_C),
            full(NODE_C, MID_C),
            full(EDGE_C, MID_C),
            full(GLOB_C, MID_C),
            full(1, MID_C),
            full(MID_C, NODE_C),
            full(1, NODE_C),
            full(NODE_C, MID_C),
            full(EDGE_C, MID_C),
            full(GLOB_C, MID_C),
            full(1, MID_C),
            full(MID_C, GLOB_C),
            full(1, GLOB_C),
        ],
        out_specs=[
            pl.BlockSpec((bn, NODE_C), lambda i: (i, 0)),
            pl.BlockSpec((N_GRAPHS, GLOB_C), lambda i: (0, 0)),
        ],
        out_shape=[
            jax.ShapeDtypeStruct((N_NODES, NODE_C), jnp.float32),
            jax.ShapeDtypeStruct((N_GRAPHS, GLOB_C), jnp.float32),
        ],
        scratch_shapes=[
            pltpu.VMEM((N_GRAPHS, MID_C), jnp.float32),
            pltpu.VMEM((N_GRAPHS, EDGE_C), jnp.float32),
        ],
    )(node, aggp, batch2d, g,
      nw1a, nw1b, nw1c, nb1, nw2, nb2,
      gw1a, gw1b, gw1c, gb1, gw2, gb2)


def kernel(node, edge, edge_index, batch, g,
           e_W1, e_b1, e_W2, e_b2,
           n_W1, n_b1, n_W2, n_b2,
           g_W1, g_b1, g_W2, g_b2):
    rol = edge_index[0]
    col = edge_index[1]
    batch2d = batch.reshape(N_NODES, 1)

    ew1a = e_W1[0:NODE_C]
    ew1b = e_W1[NODE_C:2 * NODE_C]
    ew1c = e_W1[2 * NODE_C:2 * NODE_C + EDGE_C]
    ew1d = e_W1[2 * NODE_C + EDGE_C:]
    nw1a = n_W1[0:NODE_C]
    nw1b = n_W1[NODE_C:NODE_C + EDGE_C]
    nw1c = n_W1[NODE_C + EDGE_C:]
    gw1a = g_W1[0:NODE_C]
    gw1b = g_W1[NODE_C:NODE_C + EDGE_C]
    gw1c = g_W1[NODE_C + EDGE_C:]

    pa, pb = _stage_p(node, batch2d, g, ew1a, ew1b, ew1d,
                      e_b1.reshape(1, MID_C))
    acc = _stage_g(pa, pb, rol, col)
    edge_new = _stage_e(acc, edge, ew1c, e_W2, e_b2.reshape(1, EDGE_C))
    aggp = _stage_s(rol, edge_new.reshape(N_EDGES * EDGE_C))
    aggp = aggp.reshape(NS, N_NODES, EDGE_C)
    node_new, g_new = _stage_n(
        node, aggp, batch2d, g,
        nw1a, nw1b, nw1c, n_b1.reshape(1, MID_C), n_W2,
        n_b2.reshape(1, NODE_C),
        gw1a, gw1b, gw1c, g_b1.reshape(1, MID_C), g_W2,
        g_b2.reshape(1, GLOB_C))
    return (edge_new, node_new, g_new)
